# baseline (device time: 14482 ns/iter reference)
import jax
import jax.numpy as jnp
from jax import lax
from jax.experimental import pallas as pl
from jax.experimental.pallas import tpu as pltpu

N_DEV = 32
N_Z = 4
N_P = 8


def kernel(x):
    m, n = x.shape

    def body(x_hbm, out_ref, x_vmem, zbuf, pbuf, copy_sem,
             zsend_sems, zrecv_sems, psend_sems, precv_sems, cert_p):
        my_i = lax.axis_index("i")
        my_z = my_i // N_P
        my_p = lax.rem(my_i, N_P)

        barrier_sem = pltpu.get_barrier_semaphore()
        for dz in range(1, N_Z):
            zmate = lax.rem(my_z + dz, N_Z) * N_P + my_p
            pl.semaphore_signal(
                barrier_sem, inc=1,
                device_id=(zmate,), device_id_type=pl.DeviceIdType.MESH,
            )
        for dp in range(1, N_P):
            pmate = my_z * N_P + lax.rem(my_p + dp, N_P)
            pl.semaphore_signal(
                cert_p, inc=1,
                device_id=(pmate,), device_id_type=pl.DeviceIdType.MESH,
            )

        copy = pltpu.make_async_copy(x_hbm, x_vmem, copy_sem)
        copy.start()
        copy.wait()
        partial = jnp.sum(x_vmem[:, :], axis=0, keepdims=True)
        zbuf[pl.ds(my_z, 1), :] = partial

        pl.semaphore_wait(barrier_sem, N_Z - 1)
        zsends = []
        for dz in range(1, N_Z):
            zdst = lax.rem(my_z + dz, N_Z)
            rdma = pltpu.make_async_remote_copy(
                src_ref=zbuf.at[pl.ds(my_z, 1)],
                dst_ref=zbuf.at[pl.ds(my_z, 1)],
                send_sem=zsend_sems.at[dz],
                recv_sem=zrecv_sems.at[dz],
                device_id=(zdst * N_P + my_p,),
                device_id_type=pl.DeviceIdType.MESH,
            )
            rdma.start()
            zsends.append(rdma)
        for dz in range(1, N_Z):
            zsrc = lax.rem(my_z - dz + N_Z, N_Z)
            pltpu.make_async_remote_copy(
                src_ref=zbuf.at[pl.ds(my_z, 1)],
                dst_ref=zbuf.at[pl.ds(zsrc, 1)],
                send_sem=zsend_sems.at[dz],
                recv_sem=zrecv_sems.at[dz],
                device_id=(my_i,),
                device_id_type=pl.DeviceIdType.MESH,
            ).wait_recv()

        pbuf[pl.ds(my_p, 1), :] = jnp.sum(zbuf[:, :], axis=0, keepdims=True)

        pl.semaphore_wait(cert_p, N_P - 1)
        psends = []
        for dp in range(1, N_P):
            pdst = lax.rem(my_p + dp, N_P)
            rdma = pltpu.make_async_remote_copy(
                src_ref=pbuf.at[pl.ds(my_p, 1)],
                dst_ref=pbuf.at[pl.ds(my_p, 1)],
                send_sem=psend_sems.at[dp],
                recv_sem=precv_sems.at[dp],
                device_id=(my_z * N_P + pdst,),
                device_id_type=pl.DeviceIdType.MESH,
            )
            rdma.start()
            psends.append(rdma)
        for dp in range(1, N_P):
            psrc = lax.rem(my_p - dp + N_P, N_P)
            pltpu.make_async_remote_copy(
                src_ref=pbuf.at[pl.ds(my_p, 1)],
                dst_ref=pbuf.at[pl.ds(psrc, 1)],
                send_sem=psend_sems.at[dp],
                recv_sem=precv_sems.at[dp],
                device_id=(my_i,),
                device_id_type=pl.DeviceIdType.MESH,
            ).wait_recv()

        for rdma in zsends + psends:
            rdma.wait_send()

        out_ref[:, :] = jnp.sum(pbuf[:, :], axis=0, keepdims=True)

    return pl.pallas_call(
        body,
        out_shape=jax.ShapeDtypeStruct((1, n), x.dtype),
        in_specs=[pl.BlockSpec(memory_space=pl.ANY)],
        out_specs=pl.BlockSpec(memory_space=pltpu.VMEM),
        scratch_shapes=[
            pltpu.VMEM((m, n), x.dtype),
            pltpu.VMEM((N_Z, n), x.dtype),
            pltpu.VMEM((N_P, n), x.dtype),
            pltpu.SemaphoreType.DMA,
            pltpu.SemaphoreType.DMA((N_Z,)),
            pltpu.SemaphoreType.DMA((N_Z,)),
            pltpu.SemaphoreType.DMA((N_P,)),
            pltpu.SemaphoreType.DMA((N_P,)),
            pltpu.SemaphoreType.REGULAR,
        ],
        compiler_params=pltpu.CompilerParams(collective_id=0),
    )(x)


# device time: 14372 ns/iter; 1.0077x vs baseline; 1.0077x over previous
import jax
import jax.numpy as jnp
from jax import lax
from jax.experimental import pallas as pl
from jax.experimental.pallas import tpu as pltpu

N_DEV = 32


def kernel(x):
    m, n = x.shape

    def body(x_hbm, out_ref, x_vmem, gather_buf, copy_sem, send_sems, recv_sems):
        my_i = lax.axis_index("i")

        barrier_sem = pltpu.get_barrier_semaphore()
        for d in range(1, N_DEV):
            tgt = lax.rem(my_i + d, N_DEV)
            pl.semaphore_signal(
                barrier_sem, inc=1,
                device_id=(tgt,), device_id_type=pl.DeviceIdType.MESH,
            )

        copy = pltpu.make_async_copy(x_hbm, x_vmem, copy_sem)
        copy.start()
        copy.wait()
        partial = jnp.sum(x_vmem[:, :], axis=0, keepdims=True)
        gather_buf[pl.ds(my_i, 1), :] = partial

        pl.semaphore_wait(barrier_sem, N_DEV - 1)

        sends = []
        for d in range(1, N_DEV):
            tgt = lax.rem(my_i + d, N_DEV)
            rdma = pltpu.make_async_remote_copy(
                src_ref=gather_buf.at[pl.ds(my_i, 1)],
                dst_ref=gather_buf.at[pl.ds(my_i, 1)],
                send_sem=send_sems.at[d],
                recv_sem=recv_sems.at[d],
                device_id=(tgt,),
                device_id_type=pl.DeviceIdType.MESH,
            )
            rdma.start()
            sends.append(rdma)

        for d in range(1, N_DEV):
            src = lax.rem(my_i - d + N_DEV, N_DEV)
            recv = pltpu.make_async_remote_copy(
                src_ref=gather_buf.at[pl.ds(my_i, 1)],
                dst_ref=gather_buf.at[pl.ds(src, 1)],
                send_sem=send_sems.at[d],
                recv_sem=recv_sems.at[d],
                device_id=(my_i,),
                device_id_type=pl.DeviceIdType.MESH,
            )
            recv.wait_recv()

        for rdma in sends:
            rdma.wait_send()

        out_ref[:, :] = jnp.sum(gather_buf[:, :], axis=0, keepdims=True)

    return pl.pallas_call(
        body,
        out_shape=jax.ShapeDtypeStruct((1, n), x.dtype),
        in_specs=[pl.BlockSpec(memory_space=pl.ANY)],
        out_specs=pl.BlockSpec(memory_space=pltpu.VMEM),
        scratch_shapes=[
            pltpu.VMEM((m, n), x.dtype),
            pltpu.VMEM((N_DEV, n), x.dtype),
            pltpu.SemaphoreType.DMA,
            pltpu.SemaphoreType.DMA((N_DEV,)),
            pltpu.SemaphoreType.DMA((N_DEV,)),
        ],
        compiler_params=pltpu.CompilerParams(collective_id=0),
    )(x)
